# baseline (speedup 1.0000x reference)
import math

import jax
import jax.numpy as jnp
import numpy as np
from jax import lax
from jax.experimental import pallas as pl
from jax.experimental.pallas import tpu as pltpu

from jax._src.state import utils as _state_utils

_orig_eval_bitcast_shape = _state_utils.eval_bitcast_shape


def _eval_bitcast_shape(x, dtype):
    if np.dtype(x.dtype) == np.dtype(np.bool_) and np.dtype(dtype).itemsize == 4:
        return x.shape
    if np.dtype(x.dtype).itemsize == np.dtype(dtype).itemsize:
        return x.shape
    return _orig_eval_bitcast_shape(x, dtype)


_state_utils.eval_bitcast_shape = _eval_bitcast_shape

S = 4096
E = 8
CAPACITY = math.ceil(2 * 1.25 * S / E)
CS = 256
TB = 128
GRID = S // TB
NBUF = 3


def _meta_phase(x_ref, cnt_ref, e1_s, e2_s, p1_s, p2_s, w1_s, w2_s):
    x = x_ref[...]
    xmax = jnp.max(x, axis=1, keepdims=True)
    ex = jnp.exp(x - xmax)
    logits = ex / jnp.sum(ex, axis=1, keepdims=True)

    e_iota = lax.broadcasted_iota(jnp.int32, (S, E), 1)
    m1v = jnp.max(logits, axis=1, keepdims=True)
    i1 = jnp.min(jnp.where(logits == m1v, e_iota, E), axis=1, keepdims=True)
    masked = jnp.where(e_iota == i1, -jnp.inf, logits)
    m2v = jnp.max(masked, axis=1, keepdims=True)
    i2 = jnp.min(jnp.where(masked == m2v, e_iota, E), axis=1, keepdims=True)

    m1 = (e_iota == i1).astype(jnp.float32)
    m2 = (e_iota == i2).astype(jnp.float32)

    tri = (lax.broadcasted_iota(jnp.int32, (CS, CS), 0)
           >= lax.broadcasted_iota(jnp.int32, (CS, CS), 1)).astype(jnp.float32)
    run1 = jnp.zeros((1, E), jnp.float32)
    run2 = jnp.zeros((1, E), jnp.float32)
    c1_blocks = []
    c2_blocks = []
    for j in range(S // CS):
        blk1 = m1[j * CS:(j + 1) * CS, :]
        blk2 = m2[j * CS:(j + 1) * CS, :]
        c1 = lax.dot(tri, blk1, preferred_element_type=jnp.float32) + run1
        c2 = lax.dot(tri, blk2, preferred_element_type=jnp.float32) + run2
        run1 = c1[CS - 1:CS, :]
        run2 = c2[CS - 1:CS, :]
        c1_blocks.append(c1)
        c2_blocks.append(c2)
    cum1 = jnp.concatenate(c1_blocks, axis=0)
    cum2 = jnp.concatenate(c2_blocks, axis=0)
    total1 = run1
    total2 = run2

    loc1 = cum1 - 1.0
    loc2 = cum2 - 1.0 + total1

    cap = jnp.float32(CAPACITY)
    k1 = m1 * (loc1 < cap).astype(jnp.float32)
    k2 = m2 * (loc2 < cap).astype(jnp.float32)

    e1_s[...] = i1
    e2_s[...] = i2
    p1_s[...] = jnp.sum(k1 * loc1, axis=1, keepdims=True).astype(jnp.int32)
    p2_s[...] = jnp.sum(k2 * loc2, axis=1, keepdims=True).astype(jnp.int32)
    w1_s[...] = jnp.sum(k1 * logits, axis=1, keepdims=True)
    w2_s[...] = jnp.sum(k2 * logits, axis=1, keepdims=True)
    cnt_ref[...] = (total1 + total2).astype(jnp.int32)


def _expand_common(e1_s, e2_s, p1_s, p2_s, w1_s, w2_s, tok):
    e1 = e1_s[tok, :].reshape(TB, 1, 1)
    e2 = e2_s[tok, :].reshape(TB, 1, 1)
    p1 = p1_s[tok, :].reshape(TB, 1, 1)
    p2 = p2_s[tok, :].reshape(TB, 1, 1)
    w1 = w1_s[tok, :].reshape(TB, 1, 1)
    w2 = w2_s[tok, :].reshape(TB, 1, 1)
    e_io = lax.broadcasted_iota(jnp.int32, (TB, E, 1), 1)
    we1 = jnp.where(e_io == e1, w1, 0.0)
    we2 = jnp.where(e_io == e2, w2, 0.0)
    c_io = lax.broadcasted_iota(jnp.int32, (TB, E, CAPACITY), 2)
    return jnp.where(c_io == p1, we1, 0.0) + jnp.where(c_io == p2, we2, 0.0)


def _fused_kernel(x_ref, cw_hbm, sm_hbm, cnt_ref,
                  e1_s, e2_s, p1_s, p2_s, w1_s, w2_s,
                  cw_bufs, sm_bufs, *sems):
    cw_sems = sems[:NBUF]
    sm_sems = sems[NBUF:]
    j = pl.program_id(0)

    @pl.when(j == 0)
    def _():
        _meta_phase(x_ref, cnt_ref, e1_s, e2_s, p1_s, p2_s, w1_s, w2_s)

    slot = lax.rem(j, NBUF)
    sm_hbm_i8 = sm_hbm.bitcast(jnp.int32)

    def cw_copy(blk, s):
        return pltpu.make_async_copy(
            cw_bufs.at[s], cw_hbm.at[pl.ds(blk * TB, TB)], cw_sems[s])

    def sm_copy(blk, s):
        return pltpu.make_async_copy(
            sm_bufs.at[s], sm_hbm_i8.at[pl.ds(blk * TB, TB)], sm_sems[s])

    for s in range(NBUF):
        @pl.when((j >= NBUF) & (slot == s))
        def _(s=s):
            cw_copy(j - NBUF, s).wait()
            sm_copy(j - NBUF, s).wait()

    cw = _expand_common(e1_s, e2_s, p1_s, p2_s, w1_s, w2_s, pl.ds(j * TB, TB))
    cw_bufs[slot] = cw
    sm_bufs[slot] = (cw != 0.0).astype(jnp.int32)

    for s in range(NBUF):
        @pl.when(slot == s)
        def _(s=s):
            cw_copy(j, s).start()
            sm_copy(j, s).start()

    @pl.when(j == GRID - 1)
    def _():
        for blk in range(GRID - NBUF, GRID):
            cw_copy(blk, blk % NBUF).wait()
            sm_copy(blk, blk % NBUF).wait()


@jax.jit
def kernel(inputs):
    cw, sm, cnt = pl.pallas_call(
        _fused_kernel,
        grid=(GRID,),
        in_specs=[pl.BlockSpec((S, E), lambda i: (0, 0))],
        out_shape=(
            jax.ShapeDtypeStruct((S, E, CAPACITY), jnp.float32),
            jax.ShapeDtypeStruct((S, E, CAPACITY), jnp.bool_),
            jax.ShapeDtypeStruct((1, E), jnp.int32),
        ),
        out_specs=(
            pl.BlockSpec(memory_space=pltpu.MemorySpace.HBM),
            pl.BlockSpec(memory_space=pltpu.MemorySpace.HBM),
            pl.BlockSpec((1, E), lambda i: (0, 0)),
        ),
        scratch_shapes=(
            [pltpu.VMEM((S, 1), jnp.int32)] * 4
            + [pltpu.VMEM((S, 1), jnp.float32)] * 2
            + [pltpu.VMEM((NBUF, TB, E, CAPACITY), jnp.float32),
               pltpu.VMEM((NBUF, TB, E, CAPACITY), jnp.int32)]
            + [pltpu.SemaphoreType.DMA] * (2 * NBUF)
        ),
    )(inputs)
    return (cw, sm, cnt.reshape(E))


# speedup vs baseline: 1.0019x; 1.0019x over previous
import math

import jax
import jax.numpy as jnp
import numpy as np
from jax import lax
from jax.experimental import pallas as pl
from jax.experimental.pallas import tpu as pltpu

from jax._src.state import utils as _state_utils

_orig_eval_bitcast_shape = _state_utils.eval_bitcast_shape


def _eval_bitcast_shape(x, dtype):
    if np.dtype(x.dtype) == np.dtype(np.bool_) and np.dtype(dtype).itemsize == 4:
        return x.shape
    if np.dtype(x.dtype).itemsize == np.dtype(dtype).itemsize:
        return x.shape
    return _orig_eval_bitcast_shape(x, dtype)


_state_utils.eval_bitcast_shape = _eval_bitcast_shape

S = 4096
E = 8
CAPACITY = math.ceil(2 * 1.25 * S / E)
CS = 256
TB = 128
GRID = S // TB
NBUF = 3


def _meta_phase(x_ref, cnt_ref, e1_s, e2_s, p1_s, p2_s, w1_s, w2_s):
    x = x_ref[...]
    xmax = jnp.max(x, axis=1, keepdims=True)
    ex = jnp.exp(x - xmax)
    logits = ex / jnp.sum(ex, axis=1, keepdims=True)

    e_iota = lax.broadcasted_iota(jnp.int32, (S, E), 1)
    m1v = jnp.max(logits, axis=1, keepdims=True)
    i1 = jnp.min(jnp.where(logits == m1v, e_iota, E), axis=1, keepdims=True)
    masked = jnp.where(e_iota == i1, -jnp.inf, logits)
    m2v = jnp.max(masked, axis=1, keepdims=True)
    i2 = jnp.min(jnp.where(masked == m2v, e_iota, E), axis=1, keepdims=True)

    m1 = (e_iota == i1).astype(jnp.float32)
    m2 = (e_iota == i2).astype(jnp.float32)

    tri = (lax.broadcasted_iota(jnp.int32, (CS, CS), 0)
           >= lax.broadcasted_iota(jnp.int32, (CS, CS), 1)).astype(jnp.float32)
    run1 = jnp.zeros((1, E), jnp.float32)
    run2 = jnp.zeros((1, E), jnp.float32)
    c1_blocks = []
    c2_blocks = []
    for j in range(S // CS):
        blk1 = m1[j * CS:(j + 1) * CS, :]
        blk2 = m2[j * CS:(j + 1) * CS, :]
        c1 = lax.dot(tri, blk1, preferred_element_type=jnp.float32) + run1
        c2 = lax.dot(tri, blk2, preferred_element_type=jnp.float32) + run2
        run1 = c1[CS - 1:CS, :]
        run2 = c2[CS - 1:CS, :]
        c1_blocks.append(c1)
        c2_blocks.append(c2)
    cum1 = jnp.concatenate(c1_blocks, axis=0)
    cum2 = jnp.concatenate(c2_blocks, axis=0)
    total1 = run1
    total2 = run2

    loc1 = cum1 - 1.0
    loc2 = cum2 - 1.0 + total1

    cap = jnp.float32(CAPACITY)
    k1 = m1 * (loc1 < cap).astype(jnp.float32)
    k2 = m2 * (loc2 < cap).astype(jnp.float32)

    e1_s[...] = i1
    e2_s[...] = i2
    p1_s[...] = jnp.sum(k1 * loc1, axis=1, keepdims=True).astype(jnp.int32)
    p2_s[...] = jnp.sum(k2 * loc2, axis=1, keepdims=True).astype(jnp.int32)
    w1_s[...] = jnp.sum(k1 * logits, axis=1, keepdims=True)
    w2_s[...] = jnp.sum(k2 * logits, axis=1, keepdims=True)
    cnt_ref[...] = (total1 + total2).astype(jnp.int32)


def _expand_common(e1_s, e2_s, p1_s, p2_s, w1_s, w2_s, tok):
    e1 = e1_s[tok, :].reshape(TB, 1, 1)
    e2 = e2_s[tok, :].reshape(TB, 1, 1)
    p1 = p1_s[tok, :].reshape(TB, 1, 1)
    p2 = p2_s[tok, :].reshape(TB, 1, 1)
    w1 = w1_s[tok, :].reshape(TB, 1, 1)
    w2 = w2_s[tok, :].reshape(TB, 1, 1)
    e_io = lax.broadcasted_iota(jnp.int32, (TB, E, 1), 1)
    we1 = jnp.where(e_io == e1, w1, 0.0)
    we2 = jnp.where(e_io == e2, w2, 0.0)
    c_io = lax.broadcasted_iota(jnp.int32, (TB, E, CAPACITY), 2)
    return jnp.where(c_io == p1, we1, 0.0) + jnp.where(c_io == p2, we2, 0.0)


def _fused_kernel(x_ref, cw_ref, sm_hbm, cnt_ref,
                  e1_s, e2_s, p1_s, p2_s, w1_s, w2_s,
                  sm_bufs, *sems):
    sm_sems = sems
    j = pl.program_id(0)

    @pl.when(j == 0)
    def _():
        _meta_phase(x_ref, cnt_ref, e1_s, e2_s, p1_s, p2_s, w1_s, w2_s)

    slot = lax.rem(j, NBUF)
    sm_hbm_i8 = sm_hbm.bitcast(jnp.int32)

    def sm_copy(blk, s):
        return pltpu.make_async_copy(
            sm_bufs.at[s], sm_hbm_i8.at[pl.ds(blk * TB, TB)], sm_sems[s])

    for s in range(NBUF):
        @pl.when((j >= NBUF) & (slot == s))
        def _(s=s):
            sm_copy(j - NBUF, s).wait()

    cw = _expand_common(e1_s, e2_s, p1_s, p2_s, w1_s, w2_s, pl.ds(j * TB, TB))
    cw_ref[...] = cw
    sm_bufs[slot] = (cw != 0.0).astype(jnp.int32)

    for s in range(NBUF):
        @pl.when(slot == s)
        def _(s=s):
            sm_copy(j, s).start()

    @pl.when(j == GRID - 1)
    def _():
        for blk in range(GRID - NBUF, GRID):
            sm_copy(blk, blk % NBUF).wait()


@jax.jit
def kernel(inputs):
    cw, sm, cnt = pl.pallas_call(
        _fused_kernel,
        grid=(GRID,),
        in_specs=[pl.BlockSpec((S, E), lambda i: (0, 0))],
        out_shape=(
            jax.ShapeDtypeStruct((S, E, CAPACITY), jnp.float32),
            jax.ShapeDtypeStruct((S, E, CAPACITY), jnp.bool_),
            jax.ShapeDtypeStruct((1, E), jnp.int32),
        ),
        out_specs=(
            pl.BlockSpec((TB, E, CAPACITY), lambda i: (i, 0, 0)),
            pl.BlockSpec(memory_space=pltpu.MemorySpace.HBM),
            pl.BlockSpec((1, E), lambda i: (0, 0)),
        ),
        scratch_shapes=(
            [pltpu.VMEM((S, 1), jnp.int32)] * 4
            + [pltpu.VMEM((S, 1), jnp.float32)] * 2
            + [pltpu.VMEM((NBUF, TB, E, CAPACITY), jnp.int32)]
            + [pltpu.SemaphoreType.DMA] * NBUF
        ),
    )(inputs)
    return (cw, sm, cnt.reshape(E))
